# Initial kernel scaffold; baseline (speedup 1.0000x reference)
import jax
import jax.numpy as jnp
from jax.experimental import pallas as pl


def kernel(x):
    raise NotImplementedError("write your pallas kernel here")


# speedup vs baseline: 2.9144x; 2.9144x over previous
import jax
import jax.numpy as jnp
from jax.experimental import pallas as pl
from jax.experimental.pallas import tpu as pltpu

_R = 128
_B = 512
_N = 32768


def _scan_body(x_ref, tri_ref, o_ref, carry_ref):
    i = pl.program_id(0)

    @pl.when(i == 0)
    def _():
        carry_ref[...] = jnp.zeros_like(carry_ref)

    blk = x_ref[...]
    cs = jax.lax.dot(blk, tri_ref[...], precision=jax.lax.Precision.HIGHEST)
    out = cs + carry_ref[:, 0:1]
    o_ref[...] = out
    carry_ref[...] = jnp.broadcast_to(out[:, _B - 1:_B], carry_ref.shape)


def kernel(x):
    tri = jnp.triu(jnp.ones((_B, _B), dtype=jnp.float32))
    grid = (_N // _B,)
    return pl.pallas_call(
        _scan_body,
        grid=grid,
        in_specs=[
            pl.BlockSpec((_R, _B), lambda i: (0, i)),
            pl.BlockSpec((_B, _B), lambda i: (0, 0)),
        ],
        out_specs=pl.BlockSpec((_R, _B), lambda i: (0, i)),
        out_shape=jax.ShapeDtypeStruct((_R, _N), jnp.float32),
        scratch_shapes=[pltpu.VMEM((_R, 128), jnp.float32)],
        compiler_params=pltpu.CompilerParams(
            dimension_semantics=("arbitrary",),
        ),
    )(x, tri)


# speedup vs baseline: 3.2490x; 1.1148x over previous
import jax
import jax.numpy as jnp
from jax.experimental import pallas as pl
from jax.experimental.pallas import tpu as pltpu

_R = 128
_B = 512
_C = 128
_N = 32768


def _scan_body(x_ref, tri_ref, o_ref, carry_ref):
    i = pl.program_id(0)

    @pl.when(i == 0)
    def _():
        carry_ref[...] = jnp.zeros_like(carry_ref)

    tri = tri_ref[...]
    off = carry_ref[:, 0:1]
    for c in range(_B // _C):
        blk = x_ref[:, c * _C:(c + 1) * _C]
        cs = jax.lax.dot(blk, tri, precision=jax.lax.Precision.HIGHEST)
        o_ref[:, c * _C:(c + 1) * _C] = cs + off
        off = off + cs[:, _C - 1:_C]
    carry_ref[...] = jnp.broadcast_to(off, carry_ref.shape)


def kernel(x):
    tri = jnp.triu(jnp.ones((_C, _C), dtype=jnp.float32))
    grid = (_N // _B,)
    return pl.pallas_call(
        _scan_body,
        grid=grid,
        in_specs=[
            pl.BlockSpec((_R, _B), lambda i: (0, i)),
            pl.BlockSpec((_C, _C), lambda i: (0, 0)),
        ],
        out_specs=pl.BlockSpec((_R, _B), lambda i: (0, i)),
        out_shape=jax.ShapeDtypeStruct((_R, _N), jnp.float32),
        scratch_shapes=[pltpu.VMEM((_R, 128), jnp.float32)],
        compiler_params=pltpu.CompilerParams(
            dimension_semantics=("arbitrary",),
        ),
    )(x, tri)


# speedup vs baseline: 6.6102x; 2.0345x over previous
import jax
import jax.numpy as jnp
from jax.experimental import pallas as pl
from jax.experimental.pallas import tpu as pltpu

_R = 128
_B = 2048
_C = 128
_N = 32768


def _scan_body(x_ref, tri_ref, o_ref, carry_ref):
    i = pl.program_id(0)

    @pl.when(i == 0)
    def _():
        carry_ref[...] = jnp.zeros_like(carry_ref)

    tri = tri_ref[...]
    off = carry_ref[:, 0:1]
    for c in range(_B // _C):
        blk = x_ref[:, c * _C:(c + 1) * _C]
        cs = jax.lax.dot(blk, tri, precision=jax.lax.Precision.HIGHEST)
        o_ref[:, c * _C:(c + 1) * _C] = cs + off
        off = off + cs[:, _C - 1:_C]
    carry_ref[...] = jnp.broadcast_to(off, carry_ref.shape)


def kernel(x):
    tri = jnp.triu(jnp.ones((_C, _C), dtype=jnp.float32))
    grid = (_N // _B,)
    return pl.pallas_call(
        _scan_body,
        grid=grid,
        in_specs=[
            pl.BlockSpec((_R, _B), lambda i: (0, i)),
            pl.BlockSpec((_C, _C), lambda i: (0, 0)),
        ],
        out_specs=pl.BlockSpec((_R, _B), lambda i: (0, i)),
        out_shape=jax.ShapeDtypeStruct((_R, _N), jnp.float32),
        scratch_shapes=[pltpu.VMEM((_R, 128), jnp.float32)],
        compiler_params=pltpu.CompilerParams(
            dimension_semantics=("arbitrary",),
        ),
    )(x, tri)


# speedup vs baseline: 7.9440x; 1.2018x over previous
import jax
import jax.numpy as jnp
from jax.experimental import pallas as pl
from jax.experimental.pallas import tpu as pltpu

_R = 128
_B = 4096
_C = 128
_N = 32768


def _scan_body(x_ref, tri_ref, o_ref, carry_ref):
    i = pl.program_id(0)

    @pl.when(i == 0)
    def _():
        carry_ref[...] = jnp.zeros_like(carry_ref)

    tri = tri_ref[...]
    off = carry_ref[:, 0:1]
    for c in range(_B // _C):
        blk = x_ref[:, c * _C:(c + 1) * _C]
        cs = jax.lax.dot(blk, tri, precision=jax.lax.Precision.HIGHEST)
        o_ref[:, c * _C:(c + 1) * _C] = cs + off
        off = off + cs[:, _C - 1:_C]
    carry_ref[...] = jnp.broadcast_to(off, carry_ref.shape)


def kernel(x):
    tri = jnp.triu(jnp.ones((_C, _C), dtype=jnp.float32))
    grid = (_N // _B,)
    return pl.pallas_call(
        _scan_body,
        grid=grid,
        in_specs=[
            pl.BlockSpec((_R, _B), lambda i: (0, i)),
            pl.BlockSpec((_C, _C), lambda i: (0, 0)),
        ],
        out_specs=pl.BlockSpec((_R, _B), lambda i: (0, i)),
        out_shape=jax.ShapeDtypeStruct((_R, _N), jnp.float32),
        scratch_shapes=[pltpu.VMEM((_R, 128), jnp.float32)],
        compiler_params=pltpu.CompilerParams(
            dimension_semantics=("arbitrary",),
        ),
    )(x, tri)
